# baseline (speedup 1.0000x reference)
import functools

import jax
import jax.numpy as jnp
from jax import lax
from jax.experimental import pallas as pl
from jax.experimental.pallas import tpu as pltpu
from jax.experimental.pallas import tpu_sc as plsc

CROP = 14
NPIX = CROP * CROP
NC, NS, L = 2, 16, 16
NW = NC * NS
NP = 17
CH = 128
CH14 = CH * CROP


def _sc_roialign(tbl, rec_i, rec_f, *, m, c, nhw, w):
    bpw = m // NW
    halves = c // CH

    @functools.partial(
        pl.kernel,
        out_type=jax.ShapeDtypeStruct((m, c, NPIX), jnp.float32),
        mesh=plsc.VectorSubcoreMesh(core_axis_name="c", subcore_axis_name="s"),
        scratch_types=[
            pltpu.VMEM((5, 16), jnp.int32),
            pltpu.VMEM((4, 16), jnp.float32),
            pltpu.VMEM((NP * NP * CH,), jnp.float32),
            pltpu.VMEM((NP * CH14 + 2,), jnp.float32),
            pltpu.VMEM((CH, NPIX), jnp.float32),
            pltpu.SemaphoreType.DMA,
            pltpu.SemaphoreType.DMA,
        ],
        compiler_params=pltpu.CompilerParams(needs_layout_passes=False),
    )
    def k(tbl_hbm, reci_hbm, recf_hbm, out_hbm,
          reci_v, recf_v, patch_v, xrow_v, out_v, sem, osem):
        wid = lax.axis_index("s") * NC + lax.axis_index("c")
        lane = lax.iota(jnp.int32, L)
        jmask = lane < CROP

        def box_body(bb, carry):
            mm = wid * bpw + bb
            pltpu.sync_copy(reci_hbm.at[mm], reci_v)
            pltpu.sync_copy(recf_hbm.at[mm], recf_v)
            x0rc = reci_v[0, :]
            x1rc = reci_v[1, :]
            wx0 = recf_v[0, :]
            wx1 = recf_v[1, :]
            row_base = jnp.max(reci_v[4, :])

            for half in range(halves):
                row0 = (row_base + half * nhw) * CH
                descs = [
                    pltpu.async_copy(
                        tbl_hbm.at[pl.ds(row0 + y * (w * CH), NP * CH)],
                        patch_v.at[pl.ds(y * (NP * CH), NP * CH)], sem)
                    for y in range(NP)
                ]
                for d in descs:
                    d.wait()

                def ay_body(y, cy):
                    yrow = y * (NP * CH)

                    def ac_body(cc, cx):
                        base = jnp.full((L,), yrow + cc, jnp.int32)
                        v0 = plsc.load_gather(patch_v, [base + x0rc])
                        v1 = plsc.load_gather(patch_v, [base + x1rc])
                        xrow_v[pl.ds(y * CH14 + cc * CROP, L)] = wx0 * v0 + wx1 * v1
                        return cx

                    lax.fori_loop(0, CH, ac_body, 0, unroll=False)
                    return cy

                lax.fori_loop(0, NP, ay_body, 0, unroll=False)

                def sb_body(i, ci):
                    iv = jnp.full((L,), i, jnp.int32)
                    wy0 = plsc.load_gather(recf_v, [jnp.full((L,), 2, jnp.int32), iv])
                    wy1 = plsc.load_gather(recf_v, [jnp.full((L,), 3, jnp.int32), iv])
                    y0m = jnp.max(plsc.load_gather(
                        reci_v, [jnp.full((L,), 2, jnp.int32), iv]))
                    y1m = jnp.max(plsc.load_gather(
                        reci_v, [jnp.full((L,), 3, jnp.int32), iv]))

                    ccv0 = jnp.full((L,), 0, jnp.int32)
                    jidx = jnp.full((L,), i * CROP, jnp.int32) + lane

                    def sc_body(cc, cx):
                        off = cc * CROP
                        v0 = xrow_v[pl.ds(y0m + off, L)]
                        v1 = xrow_v[pl.ds(y1m + off, L)]
                        plsc.store_scatter(
                            out_v, [ccv0 + cc, jidx], wy0 * v0 + wy1 * v1,
                            mask=jmask)
                        return cx

                    lax.fori_loop(0, CH, sc_body, 0, unroll=False)
                    return ci

                lax.fori_loop(0, CROP, sb_body, 0, unroll=False)

                pltpu.async_copy(
                    out_v, out_hbm.at[mm, pl.ds(half * CH, CH)], osem).wait()
            return carry

        lax.fori_loop(0, bpw, box_body, 0, unroll=False)

    return k(tbl, rec_i, rec_f)


def kernel(featuremap, boxes, box_ind):
    n, c, h, w = featuremap.shape
    m = boxes.shape[0]
    nhw = n * h * w
    halves = c // CH

    tblh = jnp.transpose(featuremap, (0, 2, 3, 1)).reshape(nhw, halves, CH)
    tbl = jnp.transpose(tblh, (1, 0, 2)).reshape(halves * nhw * CH)

    x1, y1, x2, y2 = boxes[:, 0], boxes[:, 1], boxes[:, 2], boxes[:, 3]
    spacing_w = (x2 - x1) / CROP
    spacing_h = (y2 - y1) / CROP
    nx0 = (x1 + spacing_w / 2 - 0.5) / (w - 1)
    ny0 = (y1 + spacing_h / 2 - 0.5) / (h - 1)
    nw_ = spacing_w * (CROP - 1) / (w - 1)
    nh_ = spacing_h * (CROP - 1) / (h - 1)
    g = jnp.linspace(0.0, 1.0, CROP)
    iy = (ny0[:, None] + nh_[:, None] * g[None, :]) * (h - 1)
    ix = (nx0[:, None] + nw_[:, None] * g[None, :]) * (w - 1)
    iy0 = jnp.floor(iy)
    ix0 = jnp.floor(ix)
    wy1 = iy - iy0
    wx1 = ix - ix0
    vy0 = (iy0 >= 0) & (iy0 <= h - 1)
    vy1 = (iy0 + 1 >= 0) & (iy0 + 1 <= h - 1)
    vx0 = (ix0 >= 0) & (ix0 <= w - 1)
    vx1 = (ix0 + 1 >= 0) & (ix0 + 1 <= w - 1)
    wy0z = (1.0 - wy1) * vy0
    wy1z = wy1 * vy1
    wx0z = (1.0 - wx1) * vx0
    wx1z = wx1 * vx1
    ix0 = ix0.astype(jnp.int32)
    iy0 = iy0.astype(jnp.int32)
    xbase = jnp.clip(ix0[:, 0], 0, w - NP)
    ybase = jnp.clip(iy0[:, 0], 0, h - NP)
    x0r = jnp.clip(ix0 - xbase[:, None], 0, NP - 1) * CH
    x1r = jnp.clip(ix0 + 1 - xbase[:, None], 0, NP - 1) * CH
    y0m = jnp.clip(iy0 - ybase[:, None], 0, NP - 1) * CH14
    y1m = jnp.clip(iy0 + 1 - ybase[:, None], 0, NP - 1) * CH14
    row_base = (box_ind.astype(jnp.int32) * h + ybase) * w + xbase

    def pad16(a):
        return jnp.pad(a, ((0, 0), (0, 16 - CROP)))

    rec_i = jnp.stack([
        pad16(x0r), pad16(x1r), pad16(y0m), pad16(y1m),
        jnp.broadcast_to(row_base[:, None], (m, 16)),
    ], axis=1).astype(jnp.int32)
    rec_f = jnp.stack(
        [pad16(wx0z), pad16(wx1z), pad16(wy0z), pad16(wy1z)], axis=1
    ).astype(jnp.float32)

    out = _sc_roialign(tbl, rec_i, rec_f, m=m, c=c, nhw=nhw, w=w)
    return out.reshape(m, c, CROP, CROP)


# speedup vs baseline: 1.0103x; 1.0103x over previous
import functools

import jax
import jax.numpy as jnp
from jax import lax
from jax.experimental import pallas as pl
from jax.experimental.pallas import tpu as pltpu
from jax.experimental.pallas import tpu_sc as plsc

CROP = 14
NPIX = CROP * CROP
NC, NS, L = 2, 16, 16
NW = NC * NS
NP = 17
CH = 128
CH14 = CH * CROP


def _sc_roialign(tbl, rec_i, rec_f, *, m, c, nhw, w):
    bpw = m // NW
    halves = c // CH

    @functools.partial(
        pl.kernel,
        out_type=jax.ShapeDtypeStruct((m, c, NPIX), jnp.float32),
        mesh=plsc.VectorSubcoreMesh(core_axis_name="c", subcore_axis_name="s"),
        scratch_types=[
            pltpu.VMEM((5, 16), jnp.int32),
            pltpu.VMEM((4, 16), jnp.float32),
            pltpu.VMEM((NP * NP * CH,), jnp.float32),
            pltpu.VMEM((NP * CH14 + 2,), jnp.float32),
            pltpu.VMEM((CH, NPIX), jnp.float32),
            pltpu.SemaphoreType.DMA,
            pltpu.SemaphoreType.DMA,
        ],
        compiler_params=pltpu.CompilerParams(needs_layout_passes=False),
    )
    def k(tbl_hbm, reci_hbm, recf_hbm, out_hbm,
          reci_v, recf_v, patch_v, xrow_v, out_v, sem, osem):
        wid = lax.axis_index("s") * NC + lax.axis_index("c")
        lane = lax.iota(jnp.int32, L)
        jmask = lane < CROP

        def box_body(bb, carry):
            mm = wid * bpw + bb
            pltpu.sync_copy(reci_hbm.at[mm], reci_v)
            pltpu.sync_copy(recf_hbm.at[mm], recf_v)
            x0rc = reci_v[0, :]
            x1rc = reci_v[1, :]
            wx0 = recf_v[0, :]
            wx1 = recf_v[1, :]
            y0m_row = reci_v[2, :]
            y1m_row = reci_v[3, :]
            wy0_row = recf_v[2, :]
            wy1_row = recf_v[3, :]
            row_base = reci_v[4, :][0]

            for half in range(halves):
                row0 = (row_base + half * nhw) * CH
                descs = [
                    pltpu.async_copy(
                        tbl_hbm.at[pl.ds(row0 + y * (w * CH), NP * CH)],
                        patch_v.at[pl.ds(y * (NP * CH), NP * CH)], sem)
                    for y in range(NP)
                ]
                for d in descs:
                    d.wait()

                def ay_body(y, cy):
                    yrow = y * (NP * CH)

                    def ac_body(cc, cx):
                        base = jnp.full((L,), yrow + cc, jnp.int32)
                        v0 = plsc.load_gather(patch_v, [base + x0rc])
                        v1 = plsc.load_gather(patch_v, [base + x1rc])
                        xrow_v[pl.ds(y * CH14 + cc * CROP, L)] = wx0 * v0 + wx1 * v1
                        return cx

                    lax.fori_loop(0, CH, ac_body, 0, unroll=8)
                    return cy

                lax.fori_loop(0, NP, ay_body, 0, unroll=False)

                for i in range(CROP):
                    wy0 = wy0_row[i]
                    wy1 = wy1_row[i]
                    y0m = y0m_row[i]
                    y1m = y1m_row[i]
                    jidx = jnp.full((L,), i * CROP, jnp.int32) + lane

                    def sc_body(cc, cx, y0m=y0m, y1m=y1m, wy0=wy0, wy1=wy1,
                                jidx=jidx):
                        off = cc * CROP
                        v0 = xrow_v[pl.ds(y0m + off, L)]
                        v1 = xrow_v[pl.ds(y1m + off, L)]
                        plsc.store_scatter(
                            out_v, [jnp.full((L,), cc, jnp.int32), jidx],
                            wy0 * v0 + wy1 * v1, mask=jmask)
                        return cx

                    lax.fori_loop(0, CH, sc_body, 0, unroll=8)

                pltpu.async_copy(
                    out_v, out_hbm.at[mm, pl.ds(half * CH, CH)], osem).wait()
            return carry

        lax.fori_loop(0, bpw, box_body, 0, unroll=False)

    return k(tbl, rec_i, rec_f)


def kernel(featuremap, boxes, box_ind):
    n, c, h, w = featuremap.shape
    m = boxes.shape[0]
    nhw = n * h * w
    halves = c // CH

    tblh = jnp.transpose(featuremap, (0, 2, 3, 1)).reshape(nhw, halves, CH)
    tbl = jnp.transpose(tblh, (1, 0, 2)).reshape(halves * nhw * CH)

    x1, y1, x2, y2 = boxes[:, 0], boxes[:, 1], boxes[:, 2], boxes[:, 3]
    spacing_w = (x2 - x1) / CROP
    spacing_h = (y2 - y1) / CROP
    nx0 = (x1 + spacing_w / 2 - 0.5) / (w - 1)
    ny0 = (y1 + spacing_h / 2 - 0.5) / (h - 1)
    nw_ = spacing_w * (CROP - 1) / (w - 1)
    nh_ = spacing_h * (CROP - 1) / (h - 1)
    g = jnp.linspace(0.0, 1.0, CROP)
    iy = (ny0[:, None] + nh_[:, None] * g[None, :]) * (h - 1)
    ix = (nx0[:, None] + nw_[:, None] * g[None, :]) * (w - 1)
    iy0 = jnp.floor(iy)
    ix0 = jnp.floor(ix)
    wy1 = iy - iy0
    wx1 = ix - ix0
    vy0 = (iy0 >= 0) & (iy0 <= h - 1)
    vy1 = (iy0 + 1 >= 0) & (iy0 + 1 <= h - 1)
    vx0 = (ix0 >= 0) & (ix0 <= w - 1)
    vx1 = (ix0 + 1 >= 0) & (ix0 + 1 <= w - 1)
    wy0z = (1.0 - wy1) * vy0
    wy1z = wy1 * vy1
    wx0z = (1.0 - wx1) * vx0
    wx1z = wx1 * vx1
    ix0 = ix0.astype(jnp.int32)
    iy0 = iy0.astype(jnp.int32)
    xbase = jnp.clip(ix0[:, 0], 0, w - NP)
    ybase = jnp.clip(iy0[:, 0], 0, h - NP)
    x0r = jnp.clip(ix0 - xbase[:, None], 0, NP - 1) * CH
    x1r = jnp.clip(ix0 + 1 - xbase[:, None], 0, NP - 1) * CH
    y0m = jnp.clip(iy0 - ybase[:, None], 0, NP - 1) * CH14
    y1m = jnp.clip(iy0 + 1 - ybase[:, None], 0, NP - 1) * CH14
    row_base = (box_ind.astype(jnp.int32) * h + ybase) * w + xbase

    def pad16(a):
        return jnp.pad(a, ((0, 0), (0, 16 - CROP)))

    rec_i = jnp.stack([
        pad16(x0r), pad16(x1r), pad16(y0m), pad16(y1m),
        jnp.broadcast_to(row_base[:, None], (m, 16)),
    ], axis=1).astype(jnp.int32)
    rec_f = jnp.stack(
        [pad16(wx0z), pad16(wx1z), pad16(wy0z), pad16(wy1z)], axis=1
    ).astype(jnp.float32)

    out = _sc_roialign(tbl, rec_i, rec_f, m=m, c=c, nhw=nhw, w=w)
    return out.reshape(m, c, CROP, CROP)


# speedup vs baseline: 2.1687x; 2.1466x over previous
import functools

import jax
import jax.numpy as jnp
from jax import lax
from jax.experimental import pallas as pl
from jax.experimental.pallas import tpu as pltpu
from jax.experimental.pallas import tpu_sc as plsc

CROP = 14
NPIX = CROP * CROP
NC, NS, L = 2, 16, 16
NW = NC * NS
NP = 17
CH = 128
XP = 15
CHXP = CH * XP


def _sc_roialign(tbl, rec_i, rec_f, *, m, c, nhw, w):
    bpw = m // NW
    halves = c // CH

    @functools.partial(
        pl.kernel,
        out_type=jax.ShapeDtypeStruct((m, c, NPIX), jnp.float32),
        mesh=plsc.VectorSubcoreMesh(core_axis_name="c", subcore_axis_name="s"),
        scratch_types=[
            pltpu.VMEM((5, 16), jnp.int32),
            pltpu.VMEM((4, 16), jnp.float32),
            pltpu.VMEM((NP * NP * CH,), jnp.float32),
            pltpu.VMEM((NP * CHXP + 16,), jnp.float32),
            pltpu.VMEM((CH, NPIX), jnp.float32),
            pltpu.SemaphoreType.DMA,
            pltpu.SemaphoreType.DMA,
        ],
        compiler_params=pltpu.CompilerParams(needs_layout_passes=False),
    )
    def k(tbl_hbm, reci_hbm, recf_hbm, out_hbm,
          reci_v, recf_v, patch_v, xrow_v, out_v, sem, osem):
        wid = lax.axis_index("s") * NC + lax.axis_index("c")
        lane = lax.iota(jnp.int32, L)
        jmask = lane < CROP
        lane_xp = lane * XP

        def box_body(bb, carry):
            mm = wid * bpw + bb
            pltpu.sync_copy(reci_hbm.at[mm], reci_v)
            pltpu.sync_copy(recf_hbm.at[mm], recf_v)
            x0rc_row = reci_v[0, :]
            x1rc_row = reci_v[1, :]
            wx0_row = recf_v[0, :]
            wx1_row = recf_v[1, :]
            y0m_row = reci_v[2, :]
            y1m_row = reci_v[3, :]
            wy0_row = recf_v[2, :]
            wy1_row = recf_v[3, :]
            row_base = reci_v[4, :][0]

            for half in range(halves):
                row0 = (row_base + half * nhw) * CH
                descs = [
                    pltpu.async_copy(
                        tbl_hbm.at[pl.ds(row0 + y * (w * CH), NP * CH)],
                        patch_v.at[pl.ds(y * (NP * CH), NP * CH)], sem)
                    for y in range(NP)
                ]
                for d in descs:
                    d.wait()

                for j in range(CROP):
                    x0c = x0rc_row[j]
                    x1c = x1rc_row[j]
                    wx0 = wx0_row[j]
                    wx1 = wx1_row[j]

                    def ay_body(y, cy, x0c=x0c, x1c=x1c, wx0=wx0, wx1=wx1,
                                j=j):
                        pbase = y * (NP * CH)
                        sbase = y * CHXP + j
                        for ch in range(CH // L):
                            v0 = patch_v[pl.ds(pbase + x0c + ch * L, L)]
                            v1 = patch_v[pl.ds(pbase + x1c + ch * L, L)]
                            plsc.store_scatter(
                                xrow_v,
                                [jnp.full((L,), sbase + ch * (L * XP),
                                          jnp.int32) + lane_xp],
                                wx0 * v0 + wx1 * v1)
                        return cy

                    lax.fori_loop(0, NP, ay_body, 0, unroll=False)

                for i in range(CROP):
                    wy0 = wy0_row[i]
                    wy1 = wy1_row[i]
                    y0m = y0m_row[i]
                    y1m = y1m_row[i]
                    jidx = jnp.full((L,), i * CROP, jnp.int32) + lane

                    def sc_body(cc, cx, y0m=y0m, y1m=y1m, wy0=wy0, wy1=wy1,
                                jidx=jidx):
                        off = cc * XP
                        v0 = xrow_v[pl.ds(y0m + off, L)]
                        v1 = xrow_v[pl.ds(y1m + off, L)]
                        plsc.store_scatter(
                            out_v, [jnp.full((L,), cc, jnp.int32), jidx],
                            wy0 * v0 + wy1 * v1, mask=jmask)
                        return cx

                    lax.fori_loop(0, CH, sc_body, 0, unroll=8)

                pltpu.async_copy(
                    out_v, out_hbm.at[mm, pl.ds(half * CH, CH)], osem).wait()
            return carry

        lax.fori_loop(0, bpw, box_body, 0, unroll=False)

    return k(tbl, rec_i, rec_f)


def kernel(featuremap, boxes, box_ind):
    n, c, h, w = featuremap.shape
    m = boxes.shape[0]
    nhw = n * h * w
    halves = c // CH

    tblh = jnp.transpose(featuremap, (0, 2, 3, 1)).reshape(nhw, halves, CH)
    tbl = jnp.transpose(tblh, (1, 0, 2)).reshape(halves * nhw * CH)

    x1, y1, x2, y2 = boxes[:, 0], boxes[:, 1], boxes[:, 2], boxes[:, 3]
    spacing_w = (x2 - x1) / CROP
    spacing_h = (y2 - y1) / CROP
    nx0 = (x1 + spacing_w / 2 - 0.5) / (w - 1)
    ny0 = (y1 + spacing_h / 2 - 0.5) / (h - 1)
    nw_ = spacing_w * (CROP - 1) / (w - 1)
    nh_ = spacing_h * (CROP - 1) / (h - 1)
    g = jnp.linspace(0.0, 1.0, CROP)
    iy = (ny0[:, None] + nh_[:, None] * g[None, :]) * (h - 1)
    ix = (nx0[:, None] + nw_[:, None] * g[None, :]) * (w - 1)
    iy0 = jnp.floor(iy)
    ix0 = jnp.floor(ix)
    wy1 = iy - iy0
    wx1 = ix - ix0
    vy0 = (iy0 >= 0) & (iy0 <= h - 1)
    vy1 = (iy0 + 1 >= 0) & (iy0 + 1 <= h - 1)
    vx0 = (ix0 >= 0) & (ix0 <= w - 1)
    vx1 = (ix0 + 1 >= 0) & (ix0 + 1 <= w - 1)
    wy0z = (1.0 - wy1) * vy0
    wy1z = wy1 * vy1
    wx0z = (1.0 - wx1) * vx0
    wx1z = wx1 * vx1
    ix0 = ix0.astype(jnp.int32)
    iy0 = iy0.astype(jnp.int32)
    xbase = jnp.clip(ix0[:, 0], 0, w - NP)
    ybase = jnp.clip(iy0[:, 0], 0, h - NP)
    x0r = jnp.clip(ix0 - xbase[:, None], 0, NP - 1) * CH
    x1r = jnp.clip(ix0 + 1 - xbase[:, None], 0, NP - 1) * CH
    y0m = jnp.clip(iy0 - ybase[:, None], 0, NP - 1) * CHXP
    y1m = jnp.clip(iy0 + 1 - ybase[:, None], 0, NP - 1) * CHXP
    row_base = (box_ind.astype(jnp.int32) * h + ybase) * w + xbase

    def pad16(a):
        return jnp.pad(a, ((0, 0), (0, 16 - CROP)))

    rec_i = jnp.stack([
        pad16(x0r), pad16(x1r), pad16(y0m), pad16(y1m),
        jnp.broadcast_to(row_base[:, None], (m, 16)),
    ], axis=1).astype(jnp.int32)
    rec_f = jnp.stack(
        [pad16(wx0z), pad16(wx1z), pad16(wy0z), pad16(wy1z)], axis=1
    ).astype(jnp.float32)

    out = _sc_roialign(tbl, rec_i, rec_f, m=m, c=c, nhw=nhw, w=w)
    return out.reshape(m, c, CROP, CROP)


# speedup vs baseline: 3.9912x; 1.8404x over previous
import functools

import jax
import jax.numpy as jnp
from jax import lax
from jax.experimental import pallas as pl
from jax.experimental.pallas import tpu as pltpu
from jax.experimental.pallas import tpu_sc as plsc

CROP = 14
NPIX = CROP * CROP
NC, NS, L = 2, 16, 16
NW = NC * NS
NP = 17
CH = 128
XP = 15
CHXP = CH * XP


def _sc_roialign(tbl, rec_i, rec_f, *, m, c, nhw, w):
    bpw = m // NW
    halves = c // CH

    @functools.partial(
        pl.kernel,
        out_type=jax.ShapeDtypeStruct((m, c, NPIX), jnp.float32),
        mesh=plsc.VectorSubcoreMesh(core_axis_name="c", subcore_axis_name="s"),
        scratch_types=[
            pltpu.VMEM((5, 16), jnp.int32),
            pltpu.VMEM((4, 16), jnp.float32),
            pltpu.VMEM((NP * NP * CH,), jnp.float32),
            pltpu.VMEM((NP * CHXP + 16,), jnp.float32),
            pltpu.VMEM((CH, NPIX), jnp.float32),
            pltpu.SemaphoreType.DMA,
            pltpu.SemaphoreType.DMA,
        ],
        compiler_params=pltpu.CompilerParams(needs_layout_passes=False),
    )
    def k(tbl_hbm, reci_hbm, recf_hbm, out_hbm,
          reci_v, recf_v, patch_v, xrow_v, out_v, sem, osem):
        wid = lax.axis_index("s") * NC + lax.axis_index("c")
        lane = lax.iota(jnp.int32, L)
        jmask = lane < CROP
        lane_xp = lane * XP

        def box_body(bb, carry):
            mm = wid * bpw + bb
            pltpu.sync_copy(reci_hbm.at[mm], reci_v)
            pltpu.sync_copy(recf_hbm.at[mm], recf_v)
            x0rc_row = reci_v[0, :]
            x1rc_row = reci_v[1, :]
            wx0_row = recf_v[0, :]
            wx1_row = recf_v[1, :]
            y0m_row = reci_v[2, :]
            y1m_row = reci_v[3, :]
            wy0_row = recf_v[2, :]
            wy1_row = recf_v[3, :]
            row_base = reci_v[4, :][0]

            for half in range(halves):
                row0 = (row_base + half * nhw) * CH
                descs = [
                    pltpu.async_copy(
                        tbl_hbm.at[pl.ds(row0 + y * (w * CH), NP * CH)],
                        patch_v.at[pl.ds(y * (NP * CH), NP * CH)], sem)
                    for y in range(NP)
                ]
                for d in descs:
                    d.wait()

                for j in range(CROP):
                    x0c = x0rc_row[j]
                    x1c = x1rc_row[j]
                    wx0 = wx0_row[j]
                    wx1 = wx1_row[j]

                    @plsc.parallel_loop(0, NP, unroll=1)
                    def ay_body(y, x0c=x0c, x1c=x1c, wx0=wx0, wx1=wx1, j=j):
                        pbase = y * (NP * CH)
                        sbase = y * CHXP + j
                        for ch in range(CH // L):
                            v0 = patch_v[pl.ds(pbase + x0c + ch * L, L)]
                            v1 = patch_v[pl.ds(pbase + x1c + ch * L, L)]
                            plsc.store_scatter(
                                xrow_v,
                                [jnp.full((L,), sbase + ch * (L * XP),
                                          jnp.int32) + lane_xp],
                                wx0 * v0 + wx1 * v1)

                for i in range(CROP):
                    wy0 = wy0_row[i]
                    wy1 = wy1_row[i]
                    y0m = y0m_row[i]
                    y1m = y1m_row[i]
                    jidx = jnp.full((L,), i * CROP, jnp.int32) + lane

                    @plsc.parallel_loop(0, CH, unroll=8)
                    def sc_body(cc, y0m=y0m, y1m=y1m, wy0=wy0, wy1=wy1,
                                jidx=jidx):
                        off = cc * XP
                        v0 = xrow_v[pl.ds(y0m + off, L)]
                        v1 = xrow_v[pl.ds(y1m + off, L)]
                        plsc.store_scatter(
                            out_v, [jnp.full((L,), cc, jnp.int32), jidx],
                            wy0 * v0 + wy1 * v1, mask=jmask)

                pltpu.async_copy(
                    out_v, out_hbm.at[mm, pl.ds(half * CH, CH)], osem).wait()
            return carry

        lax.fori_loop(0, bpw, box_body, 0, unroll=False)

    return k(tbl, rec_i, rec_f)


def kernel(featuremap, boxes, box_ind):
    n, c, h, w = featuremap.shape
    m = boxes.shape[0]
    nhw = n * h * w
    halves = c // CH

    tblh = jnp.transpose(featuremap, (0, 2, 3, 1)).reshape(nhw, halves, CH)
    tbl = jnp.transpose(tblh, (1, 0, 2)).reshape(halves * nhw * CH)

    x1, y1, x2, y2 = boxes[:, 0], boxes[:, 1], boxes[:, 2], boxes[:, 3]
    spacing_w = (x2 - x1) / CROP
    spacing_h = (y2 - y1) / CROP
    nx0 = (x1 + spacing_w / 2 - 0.5) / (w - 1)
    ny0 = (y1 + spacing_h / 2 - 0.5) / (h - 1)
    nw_ = spacing_w * (CROP - 1) / (w - 1)
    nh_ = spacing_h * (CROP - 1) / (h - 1)
    g = jnp.linspace(0.0, 1.0, CROP)
    iy = (ny0[:, None] + nh_[:, None] * g[None, :]) * (h - 1)
    ix = (nx0[:, None] + nw_[:, None] * g[None, :]) * (w - 1)
    iy0 = jnp.floor(iy)
    ix0 = jnp.floor(ix)
    wy1 = iy - iy0
    wx1 = ix - ix0
    vy0 = (iy0 >= 0) & (iy0 <= h - 1)
    vy1 = (iy0 + 1 >= 0) & (iy0 + 1 <= h - 1)
    vx0 = (ix0 >= 0) & (ix0 <= w - 1)
    vx1 = (ix0 + 1 >= 0) & (ix0 + 1 <= w - 1)
    wy0z = (1.0 - wy1) * vy0
    wy1z = wy1 * vy1
    wx0z = (1.0 - wx1) * vx0
    wx1z = wx1 * vx1
    ix0 = ix0.astype(jnp.int32)
    iy0 = iy0.astype(jnp.int32)
    xbase = jnp.clip(ix0[:, 0], 0, w - NP)
    ybase = jnp.clip(iy0[:, 0], 0, h - NP)
    x0r = jnp.clip(ix0 - xbase[:, None], 0, NP - 1) * CH
    x1r = jnp.clip(ix0 + 1 - xbase[:, None], 0, NP - 1) * CH
    y0m = jnp.clip(iy0 - ybase[:, None], 0, NP - 1) * CHXP
    y1m = jnp.clip(iy0 + 1 - ybase[:, None], 0, NP - 1) * CHXP
    row_base = (box_ind.astype(jnp.int32) * h + ybase) * w + xbase

    def pad16(a):
        return jnp.pad(a, ((0, 0), (0, 16 - CROP)))

    rec_i = jnp.stack([
        pad16(x0r), pad16(x1r), pad16(y0m), pad16(y1m),
        jnp.broadcast_to(row_base[:, None], (m, 16)),
    ], axis=1).astype(jnp.int32)
    rec_f = jnp.stack(
        [pad16(wx0z), pad16(wx1z), pad16(wy0z), pad16(wy1z)], axis=1
    ).astype(jnp.float32)

    out = _sc_roialign(tbl, rec_i, rec_f, m=m, c=c, nhw=nhw, w=w)
    return out.reshape(m, c, CROP, CROP)


# speedup vs baseline: 4.1088x; 1.0295x over previous
import functools

import jax
import jax.numpy as jnp
from jax import lax
from jax.experimental import pallas as pl
from jax.experimental.pallas import tpu as pltpu
from jax.experimental.pallas import tpu_sc as plsc

CROP = 14
NPIX = CROP * CROP
NC, NS, L = 2, 16, 16
NW = NC * NS
NP = 17
NXP = 18
CH = 64
XP = 15
CHXP = CH * XP
NPCH = NXP * CH
PWORDS = NP * NPCH
OWORDS = CH * NPIX


def _sc_roialign(tbl, rec_i, rec_f, *, m, c, nhw, w):
    bpw = m // NW
    halves = c // CH
    total = bpw * halves

    @functools.partial(
        pl.kernel,
        out_type=jax.ShapeDtypeStruct((m, c, NPIX), jnp.float32),
        mesh=plsc.VectorSubcoreMesh(core_axis_name="c", subcore_axis_name="s"),
        scratch_types=[
            pltpu.VMEM((bpw * 128,), jnp.int32),
            pltpu.VMEM((bpw * 128,), jnp.float32),
            pltpu.VMEM((2 * PWORDS,), jnp.float32),
            pltpu.VMEM((NP * CHXP + 16,), jnp.float32),
            pltpu.VMEM((2, CH, NPIX), jnp.float32),
            pltpu.SemaphoreType.DMA,
            pltpu.SemaphoreType.DMA,
        ],
        compiler_params=pltpu.CompilerParams(needs_layout_passes=False),
    )
    def k(tbl_hbm, reci_hbm, recf_hbm, out_hbm,
          reci_v, recf_v, patch_v, xrow_v, out_v, sem, osem):
        wid = lax.axis_index("s") * NC + lax.axis_index("c")
        lane = lax.iota(jnp.int32, L)
        jmask = lane < CROP
        lane_xp = lane * XP

        for bb in range(bpw):
            pltpu.sync_copy(reci_hbm.at[wid * bpw + bb],
                            reci_v.at[pl.ds(bb * 128, 128)])
            pltpu.sync_copy(recf_hbm.at[wid * bpw + bb],
                            recf_v.at[pl.ds(bb * 128, 128)])

        def slab_copies(pp, make_only):
            bx = jnp.right_shift(pp, 2)
            half = jnp.bitwise_and(pp, halves - 1)
            row_base = reci_v[pl.ds(bx * 128 + 64, L)][0]
            pbase = pl.multiple_of(jnp.bitwise_and(pp, 1) * PWORDS, 128)
            out = []
            for y in range(NP):
                src = tbl_hbm.at[pl.ds(pl.multiple_of(
                    (half * nhw + row_base + y * w) * CH, CH), NPCH)]
                dst = patch_v.at[pl.ds(pbase + y * NPCH, NPCH)]
                if make_only:
                    out.append(pltpu.make_async_copy(src, dst, sem))
                else:
                    out.append(pltpu.async_copy(src, dst, sem))
            return out

        def out_copy(pp):
            bx = jnp.right_shift(pp, 2)
            half = jnp.bitwise_and(pp, halves - 1)
            mm = wid * bpw + bx
            src = out_v.at[jnp.bitwise_and(pp, 1)]
            dst = out_hbm.at[mm, pl.ds(half * CH, CH)]
            return src, dst

        slab_copies(0, make_only=False)

        def pass_body(p, carry):
            bx = jnp.right_shift(p, 2)
            pbase = pl.multiple_of(jnp.bitwise_and(p, 1) * PWORDS, 128)
            obuf = jnp.bitwise_and(p, 1)

            @pl.when(p + 1 < total)
            def _():
                slab_copies(p + 1, make_only=False)

            for d in slab_copies(p, make_only=True):
                d.wait()

            xi0 = reci_v[pl.ds(bx * 128, L)]
            xi1 = reci_v[pl.ds(bx * 128 + 16, L)]
            yi0 = reci_v[pl.ds(bx * 128 + 32, L)]
            yi1 = reci_v[pl.ds(bx * 128 + 48, L)]
            wxf0 = recf_v[pl.ds(bx * 128, L)]
            wxf1 = recf_v[pl.ds(bx * 128 + 16, L)]
            wyf0 = recf_v[pl.ds(bx * 128 + 32, L)]
            wyf1 = recf_v[pl.ds(bx * 128 + 48, L)]
            x0c = [xi0[j] for j in range(CROP)]
            x1c = [xi1[j] for j in range(CROP)]
            y0m = [yi0[j] for j in range(CROP)]
            y1m = [yi1[j] for j in range(CROP)]
            wx0 = [wxf0[j] for j in range(CROP)]
            wx1 = [wxf1[j] for j in range(CROP)]
            wy0 = [wyf0[j] for j in range(CROP)]
            wy1 = [wyf1[j] for j in range(CROP)]

            @plsc.parallel_loop(0, NP, unroll=1)
            def ay_body(y):
                pb = y * NPCH
                sb = y * CHXP
                for j in range(CROP):
                    for ch in range(CH // L):
                        v0 = patch_v[pl.ds(pbase + pb + x0c[j] + ch * L, L)]
                        v1 = patch_v[pl.ds(pbase + pb + x1c[j] + ch * L, L)]
                        plsc.store_scatter(
                            xrow_v,
                            [jnp.full((L,), sb + j + ch * (L * XP),
                                      jnp.int32) + lane_xp],
                            wx0[j] * v0 + wx1[j] * v1)

            @pl.when(p >= 2)
            def _():
                src, dst = out_copy(p - 2)
                pltpu.make_async_copy(src, dst, osem).wait()

            obufv = jnp.full((L,), obuf, jnp.int32)

            @plsc.parallel_loop(0, CH, unroll=2)
            def sb_body(cc):
                ccv = jnp.full((L,), cc, jnp.int32)
                off = cc * XP
                for i in range(CROP):
                    v0 = xrow_v[pl.ds(y0m[i] + off, L)]
                    v1 = xrow_v[pl.ds(y1m[i] + off, L)]
                    plsc.store_scatter(
                        out_v,
                        [obufv, ccv,
                         jnp.full((L,), i * CROP, jnp.int32) + lane],
                        wy0[i] * v0 + wy1[i] * v1, mask=jmask)

            src, dst = out_copy(p)
            pltpu.async_copy(src, dst, osem)
            return carry

        lax.fori_loop(0, total, pass_body, 0, unroll=False)

        for pp in (total - 2, total - 1):
            src, dst = out_copy(jnp.int32(pp))
            pltpu.make_async_copy(src, dst, osem).wait()

    return k(tbl, rec_i, rec_f)


def kernel(featuremap, boxes, box_ind):
    n, c, h, w = featuremap.shape
    m = boxes.shape[0]
    nhw = n * h * w
    halves = c // CH

    tblh = jnp.transpose(featuremap, (0, 2, 3, 1)).reshape(nhw, halves, CH)
    tbl = jnp.transpose(tblh, (1, 0, 2)).reshape(halves * nhw * CH)

    x1, y1, x2, y2 = boxes[:, 0], boxes[:, 1], boxes[:, 2], boxes[:, 3]
    spacing_w = (x2 - x1) / CROP
    spacing_h = (y2 - y1) / CROP
    nx0 = (x1 + spacing_w / 2 - 0.5) / (w - 1)
    ny0 = (y1 + spacing_h / 2 - 0.5) / (h - 1)
    nw_ = spacing_w * (CROP - 1) / (w - 1)
    nh_ = spacing_h * (CROP - 1) / (h - 1)
    g = jnp.linspace(0.0, 1.0, CROP)
    iy = (ny0[:, None] + nh_[:, None] * g[None, :]) * (h - 1)
    ix = (nx0[:, None] + nw_[:, None] * g[None, :]) * (w - 1)
    iy0 = jnp.floor(iy)
    ix0 = jnp.floor(ix)
    wy1 = iy - iy0
    wx1 = ix - ix0
    vy0 = (iy0 >= 0) & (iy0 <= h - 1)
    vy1 = (iy0 + 1 >= 0) & (iy0 + 1 <= h - 1)
    vx0 = (ix0 >= 0) & (ix0 <= w - 1)
    vx1 = (ix0 + 1 >= 0) & (ix0 + 1 <= w - 1)
    wy0z = (1.0 - wy1) * vy0
    wy1z = wy1 * vy1
    wx0z = (1.0 - wx1) * vx0
    wx1z = wx1 * vx1
    ix0 = ix0.astype(jnp.int32)
    iy0 = iy0.astype(jnp.int32)
    xbase = jnp.clip(ix0[:, 0], 0, w - NXP)
    ybase = jnp.clip(iy0[:, 0], 0, h - NP)
    x0r = jnp.clip(ix0 - xbase[:, None], 0, NXP - 1) * CH
    x1r = jnp.clip(ix0 + 1 - xbase[:, None], 0, NXP - 1) * CH
    y0m = jnp.clip(iy0 - ybase[:, None], 0, NP - 1) * CHXP
    y1m = jnp.clip(iy0 + 1 - ybase[:, None], 0, NP - 1) * CHXP
    row_base = (box_ind.astype(jnp.int32) * h + ybase) * w + xbase

    def pad16(a):
        return jnp.pad(a, ((0, 0), (0, 16 - CROP)))

    rec_i = jnp.stack([
        pad16(x0r), pad16(x1r), pad16(y0m), pad16(y1m),
        jnp.broadcast_to(row_base[:, None], (m, 16)),
    ], axis=1).astype(jnp.int32).reshape(m, 80)
    rec_i = jnp.pad(rec_i, ((0, 0), (0, 48)))
    rec_f = jnp.stack(
        [pad16(wx0z), pad16(wx1z), pad16(wy0z), pad16(wy1z)], axis=1
    ).astype(jnp.float32).reshape(m, 64)
    rec_f = jnp.pad(rec_f, ((0, 0), (0, 64)))

    out = _sc_roialign(tbl, rec_i, rec_f, m=m, c=c, nhw=nhw, w=w)
    return out.reshape(m, c, CROP, CROP)


# speedup vs baseline: 4.4226x; 1.0764x over previous
import functools

import jax
import jax.numpy as jnp
from jax import lax
from jax.experimental import pallas as pl
from jax.experimental.pallas import tpu as pltpu
from jax.experimental.pallas import tpu_sc as plsc

CROP = 14
NPIX = CROP * CROP
NC, NS, L = 2, 16, 16
NW = NC * NS
NP = 17
NXP = 17
CH = 128
XP = 15
CHXP = CH * XP
NPCH = NXP * CH
PWORDS = NP * NPCH
OWORDS = CH * NPIX


def _sc_roialign(tbl, rec_i, rec_f, *, m, c, nhw, w):
    bpw = m // NW
    halves = c // CH
    total = bpw * halves

    @functools.partial(
        pl.kernel,
        out_type=jax.ShapeDtypeStruct((m, c, NPIX), jnp.float32),
        mesh=plsc.VectorSubcoreMesh(core_axis_name="c", subcore_axis_name="s"),
        scratch_types=[
            pltpu.VMEM((bpw * 128,), jnp.int32),
            pltpu.VMEM((bpw * 128,), jnp.float32),
            pltpu.VMEM((PWORDS,), jnp.float32),
            pltpu.VMEM((NP * CHXP + 16,), jnp.float32),
            pltpu.VMEM((CH, NPIX), jnp.float32),
            pltpu.SemaphoreType.DMA,
            pltpu.SemaphoreType.DMA,
        ],
        compiler_params=pltpu.CompilerParams(needs_layout_passes=False),
    )
    def k(tbl_hbm, reci_hbm, recf_hbm, out_hbm,
          reci_v, recf_v, patch_v, xrow_v, out_v, sem, osem):
        wid = lax.axis_index("s") * NC + lax.axis_index("c")
        lane = lax.iota(jnp.int32, L)
        jmask = lane < CROP
        lane_xp = lane * XP

        for bb in range(bpw):
            pltpu.sync_copy(reci_hbm.at[wid * bpw + bb],
                            reci_v.at[pl.ds(bb * 128, 128)])
            pltpu.sync_copy(recf_hbm.at[wid * bpw + bb],
                            recf_v.at[pl.ds(bb * 128, 128)])

        def slab_copies(pp, make_only):
            bx = jnp.right_shift(pp, 1)
            half = jnp.bitwise_and(pp, halves - 1)
            row_base = reci_v[pl.ds(bx * 128 + 64, L)][0]
            out = []
            for y in range(NP):
                src = tbl_hbm.at[pl.ds(pl.multiple_of(
                    (half * nhw + row_base + y * w) * CH, CH), NPCH)]
                dst = patch_v.at[pl.ds(y * NPCH, NPCH)]
                if make_only:
                    out.append(pltpu.make_async_copy(src, dst, sem))
                else:
                    out.append(pltpu.async_copy(src, dst, sem))
            return out

        def out_copy(pp):
            bx = jnp.right_shift(pp, 1)
            half = jnp.bitwise_and(pp, halves - 1)
            mm = wid * bpw + bx
            src = out_v
            dst = out_hbm.at[mm, pl.ds(half * CH, CH)]
            return src, dst

        slab_copies(0, make_only=False)

        def pass_body(p, carry):
            bx = jnp.right_shift(p, 1)

            for d in slab_copies(p, make_only=True):
                d.wait()

            xi0 = reci_v[pl.ds(bx * 128, L)]
            xi1 = reci_v[pl.ds(bx * 128 + 16, L)]
            yi0 = reci_v[pl.ds(bx * 128 + 32, L)]
            yi1 = reci_v[pl.ds(bx * 128 + 48, L)]
            wxf0 = recf_v[pl.ds(bx * 128, L)]
            wxf1 = recf_v[pl.ds(bx * 128 + 16, L)]
            wyf0 = recf_v[pl.ds(bx * 128 + 32, L)]
            wyf1 = recf_v[pl.ds(bx * 128 + 48, L)]
            x0c = [xi0[j] for j in range(CROP)]
            x1c = [xi1[j] for j in range(CROP)]
            y0m = [yi0[j] for j in range(CROP)]
            y1m = [yi1[j] for j in range(CROP)]
            wx0 = [wxf0[j] for j in range(CROP)]
            wx1 = [wxf1[j] for j in range(CROP)]
            wy0 = [wyf0[j] for j in range(CROP)]
            wy1 = [wyf1[j] for j in range(CROP)]

            @plsc.parallel_loop(0, NP, unroll=1)
            def ay_body(y):
                pb = y * NPCH
                sb = y * CHXP
                for j in range(CROP):
                    for ch in range(CH // L):
                        v0 = patch_v[pl.ds(pb + x0c[j] + ch * L, L)]
                        v1 = patch_v[pl.ds(pb + x1c[j] + ch * L, L)]
                        plsc.store_scatter(
                            xrow_v,
                            [jnp.full((L,), sb + j + ch * (L * XP),
                                      jnp.int32) + lane_xp],
                            wx0[j] * v0 + wx1[j] * v1)

            @pl.when(p + 1 < total)
            def _():
                slab_copies(p + 1, make_only=False)

            @pl.when(p >= 1)
            def _():
                src, dst = out_copy(p - 1)
                pltpu.make_async_copy(src, dst, osem).wait()

            @plsc.parallel_loop(0, CH, unroll=2)
            def sb_body(cc):
                ccv = jnp.full((L,), cc, jnp.int32)
                off = cc * XP
                for i in range(CROP):
                    v0 = xrow_v[pl.ds(y0m[i] + off, L)]
                    v1 = xrow_v[pl.ds(y1m[i] + off, L)]
                    plsc.store_scatter(
                        out_v,
                        [ccv,
                         jnp.full((L,), i * CROP, jnp.int32) + lane],
                        wy0[i] * v0 + wy1[i] * v1, mask=jmask)

            src, dst = out_copy(p)
            pltpu.async_copy(src, dst, osem)
            return carry

        lax.fori_loop(0, total, pass_body, 0, unroll=False)

        src, dst = out_copy(jnp.int32(total - 1))
        pltpu.make_async_copy(src, dst, osem).wait()

    return k(tbl, rec_i, rec_f)


def kernel(featuremap, boxes, box_ind):
    n, c, h, w = featuremap.shape
    m = boxes.shape[0]
    nhw = n * h * w
    halves = c // CH

    tblh = jnp.transpose(featuremap, (0, 2, 3, 1)).reshape(nhw, halves, CH)
    tbl = jnp.transpose(tblh, (1, 0, 2)).reshape(halves * nhw * CH)

    x1, y1, x2, y2 = boxes[:, 0], boxes[:, 1], boxes[:, 2], boxes[:, 3]
    spacing_w = (x2 - x1) / CROP
    spacing_h = (y2 - y1) / CROP
    nx0 = (x1 + spacing_w / 2 - 0.5) / (w - 1)
    ny0 = (y1 + spacing_h / 2 - 0.5) / (h - 1)
    nw_ = spacing_w * (CROP - 1) / (w - 1)
    nh_ = spacing_h * (CROP - 1) / (h - 1)
    g = jnp.linspace(0.0, 1.0, CROP)
    iy = (ny0[:, None] + nh_[:, None] * g[None, :]) * (h - 1)
    ix = (nx0[:, None] + nw_[:, None] * g[None, :]) * (w - 1)
    iy0 = jnp.floor(iy)
    ix0 = jnp.floor(ix)
    wy1 = iy - iy0
    wx1 = ix - ix0
    vy0 = (iy0 >= 0) & (iy0 <= h - 1)
    vy1 = (iy0 + 1 >= 0) & (iy0 + 1 <= h - 1)
    vx0 = (ix0 >= 0) & (ix0 <= w - 1)
    vx1 = (ix0 + 1 >= 0) & (ix0 + 1 <= w - 1)
    wy0z = (1.0 - wy1) * vy0
    wy1z = wy1 * vy1
    wx0z = (1.0 - wx1) * vx0
    wx1z = wx1 * vx1
    ix0 = ix0.astype(jnp.int32)
    iy0 = iy0.astype(jnp.int32)
    xbase = jnp.clip(ix0[:, 0], 0, w - NXP)
    ybase = jnp.clip(iy0[:, 0], 0, h - NP)
    x0r = jnp.clip(ix0 - xbase[:, None], 0, NXP - 1) * CH
    x1r = jnp.clip(ix0 + 1 - xbase[:, None], 0, NXP - 1) * CH
    y0m = jnp.clip(iy0 - ybase[:, None], 0, NP - 1) * CHXP
    y1m = jnp.clip(iy0 + 1 - ybase[:, None], 0, NP - 1) * CHXP
    row_base = (box_ind.astype(jnp.int32) * h + ybase) * w + xbase

    def pad16(a):
        return jnp.pad(a, ((0, 0), (0, 16 - CROP)))

    rec_i = jnp.stack([
        pad16(x0r), pad16(x1r), pad16(y0m), pad16(y1m),
        jnp.broadcast_to(row_base[:, None], (m, 16)),
    ], axis=1).astype(jnp.int32).reshape(m, 80)
    rec_i = jnp.pad(rec_i, ((0, 0), (0, 48)))
    rec_f = jnp.stack(
        [pad16(wx0z), pad16(wx1z), pad16(wy0z), pad16(wy1z)], axis=1
    ).astype(jnp.float32).reshape(m, 64)
    rec_f = jnp.pad(rec_f, ((0, 0), (0, 64)))

    out = _sc_roialign(tbl, rec_i, rec_f, m=m, c=c, nhw=nhw, w=w)
    return out.reshape(m, c, CROP, CROP)


# speedup vs baseline: 4.5017x; 1.0179x over previous
import functools

import jax
import jax.numpy as jnp
from jax import lax
from jax.experimental import pallas as pl
from jax.experimental.pallas import tpu as pltpu
from jax.experimental.pallas import tpu_sc as plsc

CROP = 14
NPIX = CROP * CROP
NC, NS, L = 2, 16, 16
NW = NC * NS
NP = 17
NXP = 17
CH = 128
XP = 15
CHXP = CH * XP
NPCH = NXP * CH
PWORDS = NP * NPCH
OWORDS = CH * NPIX


def _sc_roialign(tbl, rec_i, rec_f, *, m, c, nhw, w):
    bpw = m // NW
    halves = c // CH
    total = bpw * halves

    @functools.partial(
        pl.kernel,
        out_type=jax.ShapeDtypeStruct((m, c, NPIX), jnp.float32),
        mesh=plsc.VectorSubcoreMesh(core_axis_name="c", subcore_axis_name="s"),
        scratch_types=[
            pltpu.VMEM((bpw * 128,), jnp.int32),
            pltpu.VMEM((bpw * 128,), jnp.float32),
            pltpu.VMEM((PWORDS,), jnp.float32),
            pltpu.VMEM((NP * CHXP + 16,), jnp.float32),
            pltpu.VMEM((CH, NPIX), jnp.float32),
            pltpu.SemaphoreType.DMA,
            pltpu.SemaphoreType.DMA,
        ],
        compiler_params=pltpu.CompilerParams(needs_layout_passes=False),
    )
    def k(tbl_hbm, reci_hbm, recf_hbm, out_hbm,
          reci_v, recf_v, patch_v, xrow_v, out_v, sem, osem):
        wid = lax.axis_index("s") * NC + lax.axis_index("c")
        lane = lax.iota(jnp.int32, L)
        jmask = lane < CROP
        lane_xp = lane * XP

        for bb in range(bpw):
            pltpu.sync_copy(reci_hbm.at[wid * bpw + bb],
                            reci_v.at[pl.ds(bb * 128, 128)])
            pltpu.sync_copy(recf_hbm.at[wid * bpw + bb],
                            recf_v.at[pl.ds(bb * 128, 128)])

        def slab_copies(pp, make_only):
            bx = jnp.right_shift(pp, 1)
            half = jnp.bitwise_and(pp, halves - 1)
            row_base = reci_v[pl.ds(bx * 128 + 64, L)][0]
            out = []
            for y in range(NP):
                src = tbl_hbm.at[pl.ds(pl.multiple_of(
                    (half * nhw + row_base + y * w) * CH, CH), NPCH)]
                dst = patch_v.at[pl.ds(y * NPCH, NPCH)]
                if make_only:
                    out.append(pltpu.make_async_copy(src, dst, sem))
                else:
                    out.append(pltpu.async_copy(src, dst, sem))
            return out

        def out_copy(pp):
            bx = jnp.right_shift(pp, 1)
            half = jnp.bitwise_and(pp, halves - 1)
            mm = wid * bpw + bx
            src = out_v
            dst = out_hbm.at[mm, pl.ds(half * CH, CH)]
            return src, dst

        slab_copies(0, make_only=False)

        def pass_body(p, carry):
            bx = jnp.right_shift(p, 1)

            for d in slab_copies(p, make_only=True):
                d.wait()

            xi0 = reci_v[pl.ds(bx * 128, L)]
            xi1 = reci_v[pl.ds(bx * 128 + 16, L)]
            yi0 = reci_v[pl.ds(bx * 128 + 32, L)]
            yi1 = reci_v[pl.ds(bx * 128 + 48, L)]
            wxf0 = recf_v[pl.ds(bx * 128, L)]
            wxf1 = recf_v[pl.ds(bx * 128 + 16, L)]
            wyf0 = recf_v[pl.ds(bx * 128 + 32, L)]
            wyf1 = recf_v[pl.ds(bx * 128 + 48, L)]
            x0c = [xi0[j] for j in range(CROP)]
            x1c = [xi1[j] for j in range(CROP)]
            y0m = [yi0[j] for j in range(CROP)]
            y1m = [yi1[j] for j in range(CROP)]
            wx0 = [wxf0[j] for j in range(CROP)]
            wx1 = [wxf1[j] for j in range(CROP)]
            wy0 = [wyf0[j] for j in range(CROP)]
            wy1 = [wyf1[j] for j in range(CROP)]

            @plsc.parallel_loop(0, NP, unroll=1)
            def ay_body(y):
                pb = y * NPCH
                sb = y * CHXP
                for j in range(CROP):
                    for ch in range(CH // L):
                        v0 = patch_v[pl.ds(pb + x0c[j] + ch * L, L)]
                        v1 = patch_v[pl.ds(pb + x1c[j] + ch * L, L)]
                        plsc.store_scatter(
                            xrow_v,
                            [jnp.full((L,), sb + j + ch * (L * XP),
                                      jnp.int32) + lane_xp],
                            wx0[j] * v0 + wx1[j] * v1)

            @pl.when(p + 1 < total)
            def _():
                slab_copies(p + 1, make_only=False)

            @pl.when(p >= 1)
            def _():
                src, dst = out_copy(p - 1)
                pltpu.make_async_copy(src, dst, osem).wait()

            @plsc.parallel_loop(0, CH, unroll=4)
            def sb_body(cc):
                ccv = jnp.full((L,), cc, jnp.int32)
                off = cc * XP
                for i in range(CROP):
                    v0 = xrow_v[pl.ds(y0m[i] + off, L)]
                    v1 = xrow_v[pl.ds(y1m[i] + off, L)]
                    plsc.store_scatter(
                        out_v,
                        [ccv,
                         jnp.full((L,), i * CROP, jnp.int32) + lane],
                        wy0[i] * v0 + wy1[i] * v1, mask=jmask)

            src, dst = out_copy(p)
            pltpu.async_copy(src, dst, osem)
            return carry

        lax.fori_loop(0, total, pass_body, 0, unroll=False)

        src, dst = out_copy(jnp.int32(total - 1))
        pltpu.make_async_copy(src, dst, osem).wait()

    return k(tbl, rec_i, rec_f)


def kernel(featuremap, boxes, box_ind):
    n, c, h, w = featuremap.shape
    m = boxes.shape[0]
    nhw = n * h * w
    halves = c // CH

    tblh = jnp.transpose(featuremap, (0, 2, 3, 1)).reshape(nhw, halves, CH)
    tbl = jnp.transpose(tblh, (1, 0, 2)).reshape(halves * nhw * CH)

    x1, y1, x2, y2 = boxes[:, 0], boxes[:, 1], boxes[:, 2], boxes[:, 3]
    spacing_w = (x2 - x1) / CROP
    spacing_h = (y2 - y1) / CROP
    nx0 = (x1 + spacing_w / 2 - 0.5) / (w - 1)
    ny0 = (y1 + spacing_h / 2 - 0.5) / (h - 1)
    nw_ = spacing_w * (CROP - 1) / (w - 1)
    nh_ = spacing_h * (CROP - 1) / (h - 1)
    g = jnp.linspace(0.0, 1.0, CROP)
    iy = (ny0[:, None] + nh_[:, None] * g[None, :]) * (h - 1)
    ix = (nx0[:, None] + nw_[:, None] * g[None, :]) * (w - 1)
    iy0 = jnp.floor(iy)
    ix0 = jnp.floor(ix)
    wy1 = iy - iy0
    wx1 = ix - ix0
    vy0 = (iy0 >= 0) & (iy0 <= h - 1)
    vy1 = (iy0 + 1 >= 0) & (iy0 + 1 <= h - 1)
    vx0 = (ix0 >= 0) & (ix0 <= w - 1)
    vx1 = (ix0 + 1 >= 0) & (ix0 + 1 <= w - 1)
    wy0z = (1.0 - wy1) * vy0
    wy1z = wy1 * vy1
    wx0z = (1.0 - wx1) * vx0
    wx1z = wx1 * vx1
    ix0 = ix0.astype(jnp.int32)
    iy0 = iy0.astype(jnp.int32)
    xbase = jnp.clip(ix0[:, 0], 0, w - NXP)
    ybase = jnp.clip(iy0[:, 0], 0, h - NP)
    x0r = jnp.clip(ix0 - xbase[:, None], 0, NXP - 1) * CH
    x1r = jnp.clip(ix0 + 1 - xbase[:, None], 0, NXP - 1) * CH
    y0m = jnp.clip(iy0 - ybase[:, None], 0, NP - 1) * CHXP
    y1m = jnp.clip(iy0 + 1 - ybase[:, None], 0, NP - 1) * CHXP
    row_base = (box_ind.astype(jnp.int32) * h + ybase) * w + xbase

    def pad16(a):
        return jnp.pad(a, ((0, 0), (0, 16 - CROP)))

    rec_i = jnp.stack([
        pad16(x0r), pad16(x1r), pad16(y0m), pad16(y1m),
        jnp.broadcast_to(row_base[:, None], (m, 16)),
    ], axis=1).astype(jnp.int32).reshape(m, 80)
    rec_i = jnp.pad(rec_i, ((0, 0), (0, 48)))
    rec_f = jnp.stack(
        [pad16(wx0z), pad16(wx1z), pad16(wy0z), pad16(wy1z)], axis=1
    ).astype(jnp.float32).reshape(m, 64)
    rec_f = jnp.pad(rec_f, ((0, 0), (0, 64)))

    out = _sc_roialign(tbl, rec_i, rec_f, m=m, c=c, nhw=nhw, w=w)
    return out.reshape(m, c, CROP, CROP)


# speedup vs baseline: 4.5463x; 1.0099x over previous
import functools

import jax
import jax.numpy as jnp
from jax import lax
from jax.experimental import pallas as pl
from jax.experimental.pallas import tpu as pltpu
from jax.experimental.pallas import tpu_sc as plsc

CROP = 14
NPIX = CROP * CROP
NC, NS, L = 2, 16, 16
NW = NC * NS
NP = 17
NXP = 17
CH = 128
XP = 15
CHXP = CH * XP
NPCH = NXP * CH
PWORDS = NP * NPCH
OWORDS = CH * NPIX


def _sc_roialign(tbl, rec_i, rec_f, *, m, c, nhw, w):
    bpw = m // NW
    halves = c // CH
    total = bpw * halves

    @functools.partial(
        pl.kernel,
        out_type=jax.ShapeDtypeStruct((m, c, NPIX), jnp.float32),
        mesh=plsc.VectorSubcoreMesh(core_axis_name="c", subcore_axis_name="s"),
        scratch_types=[
            pltpu.VMEM((bpw * 128,), jnp.int32),
            pltpu.VMEM((bpw * 128,), jnp.float32),
            pltpu.VMEM((PWORDS,), jnp.float32),
            pltpu.VMEM((NP * CHXP + 16,), jnp.float32),
            pltpu.VMEM((CH, NPIX), jnp.float32),
            pltpu.SemaphoreType.DMA,
            pltpu.SemaphoreType.DMA,
        ],
        compiler_params=pltpu.CompilerParams(needs_layout_passes=False),
    )
    def k(tbl_hbm, reci_hbm, recf_hbm, out_hbm,
          reci_v, recf_v, patch_v, xrow_v, out_v, sem, osem):
        wid = lax.axis_index("s") * NC + lax.axis_index("c")
        lane = lax.iota(jnp.int32, L)
        jmask = lane < CROP
        lane_xp = lane * XP

        for bb in range(bpw):
            pltpu.sync_copy(reci_hbm.at[wid * bpw + bb],
                            reci_v.at[pl.ds(bb * 128, 128)])
            pltpu.sync_copy(recf_hbm.at[wid * bpw + bb],
                            recf_v.at[pl.ds(bb * 128, 128)])

        def slab_copies(pp, make_only):
            bx = jnp.right_shift(pp, 1)
            half = jnp.bitwise_and(pp, halves - 1)
            row_base = reci_v[pl.ds(bx * 128 + 64, L)][0]
            out = []
            for y in range(NP):
                src = tbl_hbm.at[pl.ds(pl.multiple_of(
                    (half * nhw + row_base + y * w) * CH, CH), NPCH)]
                dst = patch_v.at[pl.ds(y * NPCH, NPCH)]
                if make_only:
                    out.append(pltpu.make_async_copy(src, dst, sem))
                else:
                    out.append(pltpu.async_copy(src, dst, sem))
            return out

        def out_copy(pp):
            bx = jnp.right_shift(pp, 1)
            half = jnp.bitwise_and(pp, halves - 1)
            mm = wid * bpw + bx
            src = out_v
            dst = out_hbm.at[mm, pl.ds(half * CH, CH)]
            return src, dst

        slab_copies(0, make_only=False)

        def pass_body(p, carry):
            bx = jnp.right_shift(p, 1)

            pltpu.make_async_copy(
                tbl_hbm.at[pl.ds(0, PWORDS)], patch_v, sem).wait()

            xi0 = reci_v[pl.ds(bx * 128, L)]
            xi1 = reci_v[pl.ds(bx * 128 + 16, L)]
            yi0 = reci_v[pl.ds(bx * 128 + 32, L)]
            yi1 = reci_v[pl.ds(bx * 128 + 48, L)]
            wxf0 = recf_v[pl.ds(bx * 128, L)]
            wxf1 = recf_v[pl.ds(bx * 128 + 16, L)]
            wyf0 = recf_v[pl.ds(bx * 128 + 32, L)]
            wyf1 = recf_v[pl.ds(bx * 128 + 48, L)]
            x0c = [xi0[j] for j in range(CROP)]
            x1c = [xi1[j] for j in range(CROP)]
            y0m = [yi0[j] for j in range(CROP)]
            y1m = [yi1[j] for j in range(CROP)]
            wx0 = [wxf0[j] for j in range(CROP)]
            wx1 = [wxf1[j] for j in range(CROP)]
            wy0 = [wyf0[j] for j in range(CROP)]
            wy1 = [wyf1[j] for j in range(CROP)]

            @plsc.parallel_loop(0, NP, unroll=1)
            def ay_body(y):
                pb = y * NPCH
                sb = y * CHXP
                for j in range(CROP):
                    for ch in range(CH // L):
                        v0 = patch_v[pl.ds(pb + x0c[j] + ch * L, L)]
                        v1 = patch_v[pl.ds(pb + x1c[j] + ch * L, L)]
                        plsc.store_scatter(
                            xrow_v,
                            [jnp.full((L,), sb + j + ch * (L * XP),
                                      jnp.int32) + lane_xp],
                            wx0[j] * v0 + wx1[j] * v1)

            @pl.when(p + 1 < total)
            def _():
                slab_copies(p + 1, make_only=False)

            @pl.when(p >= 1)
            def _():
                src, dst = out_copy(p - 1)
                pltpu.make_async_copy(src, dst, osem).wait()

            @plsc.parallel_loop(0, CH, unroll=4)
            def sb_body(cc):
                ccv = jnp.full((L,), cc, jnp.int32)
                off = cc * XP
                for i in range(CROP):
                    v0 = xrow_v[pl.ds(y0m[i] + off, L)]
                    v1 = xrow_v[pl.ds(y1m[i] + off, L)]
                    plsc.store_scatter(
                        out_v,
                        [ccv,
                         jnp.full((L,), i * CROP, jnp.int32) + lane],
                        wy0[i] * v0 + wy1[i] * v1, mask=jmask)

            src, dst = out_copy(p)
            pltpu.async_copy(src, dst, osem)
            return carry

        lax.fori_loop(0, total, pass_body, 0, unroll=False)

        src, dst = out_copy(jnp.int32(total - 1))
        pltpu.make_async_copy(src, dst, osem).wait()

    return k(tbl, rec_i, rec_f)


def kernel(featuremap, boxes, box_ind):
    n, c, h, w = featuremap.shape
    m = boxes.shape[0]
    nhw = n * h * w
    halves = c // CH

    tblh = jnp.transpose(featuremap, (0, 2, 3, 1)).reshape(nhw, halves, CH)
    tbl = jnp.transpose(tblh, (1, 0, 2)).reshape(halves * nhw * CH)

    x1, y1, x2, y2 = boxes[:, 0], boxes[:, 1], boxes[:, 2], boxes[:, 3]
    spacing_w = (x2 - x1) / CROP
    spacing_h = (y2 - y1) / CROP
    nx0 = (x1 + spacing_w / 2 - 0.5) / (w - 1)
    ny0 = (y1 + spacing_h / 2 - 0.5) / (h - 1)
    nw_ = spacing_w * (CROP - 1) / (w - 1)
    nh_ = spacing_h * (CROP - 1) / (h - 1)
    g = jnp.linspace(0.0, 1.0, CROP)
    iy = (ny0[:, None] + nh_[:, None] * g[None, :]) * (h - 1)
    ix = (nx0[:, None] + nw_[:, None] * g[None, :]) * (w - 1)
    iy0 = jnp.floor(iy)
    ix0 = jnp.floor(ix)
    wy1 = iy - iy0
    wx1 = ix - ix0
    vy0 = (iy0 >= 0) & (iy0 <= h - 1)
    vy1 = (iy0 + 1 >= 0) & (iy0 + 1 <= h - 1)
    vx0 = (ix0 >= 0) & (ix0 <= w - 1)
    vx1 = (ix0 + 1 >= 0) & (ix0 + 1 <= w - 1)
    wy0z = (1.0 - wy1) * vy0
    wy1z = wy1 * vy1
    wx0z = (1.0 - wx1) * vx0
    wx1z = wx1 * vx1
    ix0 = ix0.astype(jnp.int32)
    iy0 = iy0.astype(jnp.int32)
    xbase = jnp.clip(ix0[:, 0], 0, w - NXP)
    ybase = jnp.clip(iy0[:, 0], 0, h - NP)
    x0r = jnp.clip(ix0 - xbase[:, None], 0, NXP - 1) * CH
    x1r = jnp.clip(ix0 + 1 - xbase[:, None], 0, NXP - 1) * CH
    y0m = jnp.clip(iy0 - ybase[:, None], 0, NP - 1) * CHXP
    y1m = jnp.clip(iy0 + 1 - ybase[:, None], 0, NP - 1) * CHXP
    row_base = (box_ind.astype(jnp.int32) * h + ybase) * w + xbase

    def pad16(a):
        return jnp.pad(a, ((0, 0), (0, 16 - CROP)))

    rec_i = jnp.stack([
        pad16(x0r), pad16(x1r), pad16(y0m), pad16(y1m),
        jnp.broadcast_to(row_base[:, None], (m, 16)),
    ], axis=1).astype(jnp.int32).reshape(m, 80)
    rec_i = jnp.pad(rec_i, ((0, 0), (0, 48)))
    rec_f = jnp.stack(
        [pad16(wx0z), pad16(wx1z), pad16(wy0z), pad16(wy1z)], axis=1
    ).astype(jnp.float32).reshape(m, 64)
    rec_f = jnp.pad(rec_f, ((0, 0), (0, 64)))

    out = _sc_roialign(tbl, rec_i, rec_f, m=m, c=c, nhw=nhw, w=w)
    return out.reshape(m, c, CROP, CROP)
